# baseline (speedup 1.0000x reference)
import functools

import jax
import jax.numpy as jnp
from jax import lax
from jax.experimental import pallas as pl
from jax.experimental.pallas import tpu as pltpu

_ZERO_WEIGHTING = 0.5
_NONZERO_WEIGHTING = 1.0

_PLANES = 384
_H = 224
_W = 224
_BLOCK_PLANES = 32
_GRID = _PLANES // _BLOCK_PLANES
_SLAB_ROWS = 56


_SUBS = 4
_SUB_PLANES = _BLOCK_PLANES // _SUBS


def _copies(rec_hbm, tgt_hbm, rec_v, tgt_v, sems, i, buf, s):
    p0 = i * _BLOCK_PLANES + s * _SUB_PLANES
    v0 = s * _SUB_PLANES
    out = []
    for k, (hbm, vmem) in enumerate(((rec_hbm, rec_v), (tgt_hbm, tgt_v))):
        out.append(
            pltpu.make_async_copy(
                hbm.at[pl.ds(p0, _SUB_PLANES), :, pl.ds(0, 128)],
                vmem.at[buf, pl.ds(v0, _SUB_PLANES), :, pl.ds(0, 128)],
                sems.at[buf, s, 2 * k],
            )
        )
        out.append(
            pltpu.make_async_copy(
                hbm.at[pl.ds(p0, _SUB_PLANES), :, pl.ds(128, _W - 128)],
                vmem.at[buf, pl.ds(v0, _SUB_PLANES), :, pl.ds(128, _W - 128)],
                sems.at[buf, s, 2 * k + 1],
            )
        )
    return out


def _loss_body(rec_hbm, tgt_hbm, out_ref, acc_ref, rec_v, tgt_v, sems, *, total_n):
    i = pl.program_id(0)
    n = pl.num_programs(0)
    buf = lax.rem(i, 3)

    @pl.when(i == 0)
    def _prologue():
        for blk in (0, 1):
            for s in range(_SUBS):
                for c in _copies(rec_hbm, tgt_hbm, rec_v, tgt_v, sems, blk, blk, s):
                    c.start()

    @pl.when(i + 2 < n)
    def _prefetch():
        nbuf = lax.rem(i + 2, 3)
        for s in range(_SUBS):
            for c in _copies(rec_hbm, tgt_hbm, rec_v, tgt_v, sems, i + 2, nbuf, s):
                c.start()

    def slab(p, r0, carry):
        a1, a2, a3 = carry
        t = tgt_v[buf, p, pl.ds(r0, _SLAB_ROWS), :]
        r = rec_v[buf, p, pl.ds(r0, _SLAB_ROWS), :]
        zero = t == 0.0
        d = r - t
        s1 = jnp.where(zero, d * d, 0.0).reshape(_SLAB_ROWS // 8, 8, _W).sum(axis=0)
        s2 = (
            jnp.where(zero, 0.0, jnp.abs(d))
            .reshape(_SLAB_ROWS // 8, 8, _W)
            .sum(axis=0)
        )
        s3 = jnp.where(zero, 1.0, 0.0).reshape(_SLAB_ROWS // 8, 8, _W).sum(axis=0)
        return (a1 + s1, a2 + s2, a3 + s3)

    def plane(p, carry):
        for q in range(_H // _SLAB_ROWS):
            carry = slab(p, q * _SLAB_ROWS, carry)
        return carry

    z8 = jnp.zeros((8, _W), jnp.float32)
    carry = (z8, z8, z8)
    for s in range(_SUBS):
        for c in _copies(rec_hbm, tgt_hbm, rec_v, tgt_v, sems, i, buf, s):
            c.wait()
        carry = lax.fori_loop(
            s * _SUB_PLANES, (s + 1) * _SUB_PLANES, plane, carry
        )
    a1, a2, a3 = carry
    ssq = jnp.sum(a1)
    sab = jnp.sum(a2)
    nz = jnp.sum(a3)

    @pl.when(i == 0)
    def _init():
        acc_ref[0] = 0.0
        acc_ref[1] = 0.0
        acc_ref[2] = 0.0

    acc_ref[0] += ssq
    acc_ref[1] += sab
    acc_ref[2] += nz

    @pl.when(i == n - 1)
    def _finish():
        n_zero = acc_ref[2]
        n_nonzero = total_n - n_zero
        zero_loss = jnp.where(n_zero > 0, acc_ref[0] / jnp.maximum(n_zero, 1.0), 0.0)
        nonzero_loss = jnp.where(
            n_nonzero > 0, acc_ref[1] / jnp.maximum(n_nonzero, 1.0), 0.0
        )
        out_ref[0] = _ZERO_WEIGHTING * zero_loss + _NONZERO_WEIGHTING * nonzero_loss


def kernel(reconstructed_image, target_image):
    total_n = float(reconstructed_image.size)
    rec = reconstructed_image.reshape(_PLANES, _H, _W)
    tgt = target_image.reshape(_PLANES, _H, _W)

    out = pl.pallas_call(
        functools.partial(_loss_body, total_n=total_n),
        grid=(_GRID,),
        in_specs=[
            pl.BlockSpec(memory_space=pl.ANY),
            pl.BlockSpec(memory_space=pl.ANY),
        ],
        out_specs=pl.BlockSpec(memory_space=pltpu.SMEM),
        out_shape=jax.ShapeDtypeStruct((1,), jnp.float32),
        scratch_shapes=[
            pltpu.SMEM((3,), jnp.float32),
            pltpu.VMEM((3, _BLOCK_PLANES, _H, _W), jnp.float32),
            pltpu.VMEM((3, _BLOCK_PLANES, _H, _W), jnp.float32),
            pltpu.SemaphoreType.DMA((3, _SUBS, 4)),
        ],
    )(rec, tgt)
    return out[0]


# speedup vs baseline: 1.0067x; 1.0067x over previous
import functools

import jax
import jax.numpy as jnp
from jax import lax
from jax.experimental import pallas as pl
from jax.experimental.pallas import tpu as pltpu

_ZERO_WEIGHTING = 0.5
_NONZERO_WEIGHTING = 1.0

_PLANES = 384
_H = 224
_W = 224
_BLOCK_PLANES = 32
_GRID = _PLANES // _BLOCK_PLANES
_SLAB_ROWS = 56


_SUBS = 4
_SUB_PLANES = _BLOCK_PLANES // _SUBS


def _copies(rec_hbm, tgt_hbm, rec_v, tgt_v, sems, i, buf, s):
    p0 = i * _BLOCK_PLANES + s * _SUB_PLANES
    v0 = s * _SUB_PLANES
    out = []
    for k, (hbm, vmem) in enumerate(((rec_hbm, rec_v), (tgt_hbm, tgt_v))):
        out.append(
            pltpu.make_async_copy(
                hbm.at[pl.ds(p0, _SUB_PLANES), :, pl.ds(0, 128)],
                vmem.at[buf, pl.ds(v0, _SUB_PLANES), :, pl.ds(0, 128)],
                sems.at[buf, s, 2 * k],
            )
        )
        out.append(
            pltpu.make_async_copy(
                hbm.at[pl.ds(p0, _SUB_PLANES), :, pl.ds(128, _W - 128)],
                vmem.at[buf, pl.ds(v0, _SUB_PLANES), :, pl.ds(128, _W - 128)],
                sems.at[buf, s, 2 * k + 1],
            )
        )
    return out


def _loss_body(rec_hbm, tgt_hbm, out_ref, acc_ref, rec_v, tgt_v, sems, *, total_n):
    i = pl.program_id(0)
    n = pl.num_programs(0)
    buf = lax.rem(i, 2)

    @pl.when(i == 0)
    def _prologue():
        for s in range(_SUBS):
            for c in _copies(rec_hbm, tgt_hbm, rec_v, tgt_v, sems, 0, 0, s):
                c.start()

    @pl.when(i + 1 < n)
    def _prefetch():
        for s in range(_SUBS):
            for c in _copies(rec_hbm, tgt_hbm, rec_v, tgt_v, sems, i + 1, 1 - buf, s):
                c.start()

    def slab(p, r0, carry):
        a1, a2, a3 = carry
        t = tgt_v[buf, p, pl.ds(r0, _SLAB_ROWS), :]
        r = rec_v[buf, p, pl.ds(r0, _SLAB_ROWS), :]
        zero = t == 0.0
        d = r - t
        s1 = jnp.where(zero, d * d, 0.0).reshape(_SLAB_ROWS // 8, 8, _W).sum(axis=0)
        s2 = (
            jnp.where(zero, 0.0, jnp.abs(d))
            .reshape(_SLAB_ROWS // 8, 8, _W)
            .sum(axis=0)
        )
        s3 = jnp.where(zero, 1.0, 0.0).reshape(_SLAB_ROWS // 8, 8, _W).sum(axis=0)
        return (a1 + s1, a2 + s2, a3 + s3)

    def plane(p, carry):
        for q in range(_H // _SLAB_ROWS):
            carry = slab(p, q * _SLAB_ROWS, carry)
        return carry

    z8 = jnp.zeros((8, _W), jnp.float32)
    carry = (z8, z8, z8)
    for s in range(_SUBS):
        for c in _copies(rec_hbm, tgt_hbm, rec_v, tgt_v, sems, i, buf, s):
            c.wait()
        carry = lax.fori_loop(
            s * _SUB_PLANES, (s + 1) * _SUB_PLANES, plane, carry
        )
    a1, a2, a3 = carry
    ssq = jnp.sum(a1)
    sab = jnp.sum(a2)
    nz = jnp.sum(a3)

    @pl.when(i == 0)
    def _init():
        acc_ref[0] = 0.0
        acc_ref[1] = 0.0
        acc_ref[2] = 0.0

    acc_ref[0] += ssq
    acc_ref[1] += sab
    acc_ref[2] += nz

    @pl.when(i == n - 1)
    def _finish():
        n_zero = acc_ref[2]
        n_nonzero = total_n - n_zero
        zero_loss = jnp.where(n_zero > 0, acc_ref[0] / jnp.maximum(n_zero, 1.0), 0.0)
        nonzero_loss = jnp.where(
            n_nonzero > 0, acc_ref[1] / jnp.maximum(n_nonzero, 1.0), 0.0
        )
        out_ref[0] = _ZERO_WEIGHTING * zero_loss + _NONZERO_WEIGHTING * nonzero_loss


def kernel(reconstructed_image, target_image):
    total_n = float(reconstructed_image.size)
    rec = reconstructed_image.reshape(_PLANES, _H, _W)
    tgt = target_image.reshape(_PLANES, _H, _W)

    out = pl.pallas_call(
        functools.partial(_loss_body, total_n=total_n),
        grid=(_GRID,),
        in_specs=[
            pl.BlockSpec(memory_space=pl.ANY),
            pl.BlockSpec(memory_space=pl.ANY),
        ],
        out_specs=pl.BlockSpec(memory_space=pltpu.SMEM),
        out_shape=jax.ShapeDtypeStruct((1,), jnp.float32),
        scratch_shapes=[
            pltpu.SMEM((3,), jnp.float32),
            pltpu.VMEM((2, _BLOCK_PLANES, _H, _W), jnp.float32),
            pltpu.VMEM((2, _BLOCK_PLANES, _H, _W), jnp.float32),
            pltpu.SemaphoreType.DMA((2, _SUBS, 4)),
        ],
    )(rec, tgt)
    return out[0]
